# baseline (speedup 1.0000x reference)
import functools

import jax
import jax.numpy as jnp
from jax import lax
from jax.experimental import pallas as pl
from jax.experimental.pallas import tpu as pltpu
from jax.experimental.pallas import tpu_sc as plsc

N = 10000
E = 320000
C = 128
DT = 256
EPS = 1e-5

NC = 2
NS = 16
NW = NC * NS
B = 128
CH = 80
TPW = CH * B
EPAD = NW * TPW
NPAD = 10240
RPS = NPAD // NS
DEGW = 16
RB = 1024

_MESH = plsc.VectorSubcoreMesh(
    core_axis_name="c", subcore_axis_name="s", num_cores=NC, num_subcores=NS)


@functools.partial(
    pl.kernel,
    out_type=jax.ShapeDtypeStruct((NC * NPAD, DEGW), jnp.float32),
    mesh=_MESH,
    scratch_types=[
        pltpu.VMEM((CH, B), jnp.int32),
        pltpu.VMEM((B, DEGW), jnp.float32),
        pltpu.VMEM((B, DEGW), jnp.float32),
        pltpu.VMEM_SHARED((NPAD, DEGW), jnp.float32),
    ],
)
def _deg_kernel(dst_hbm, out_hbm, dst_v, ones_v, zbuf_v, deg_sh):
    cid = lax.axis_index("c")
    sid = lax.axis_index("s")
    wid = cid * NS + sid
    pltpu.sync_copy(dst_hbm.at[pl.ds(wid * CH, CH)], dst_v)
    lane = lax.iota(jnp.int32, 16)
    one0 = jnp.where(lane == 0, 1.0, 0.0).astype(jnp.float32)
    zv = jnp.zeros((16,), jnp.float32)
    for r in range(B):
        ones_v[r, :] = one0
        zbuf_v[r, :] = zv
    for t in range(RPS // B):
        pltpu.sync_copy(zbuf_v, deg_sh.at[pl.ds(sid * RPS + t * B, B)])
    plsc.subcore_barrier()

    def body(j, carry):
        pltpu.sync_copy(ones_v, deg_sh.at[dst_v.at[j]], add=True)
        return carry

    lax.fori_loop(0, CH, body, 0)
    plsc.subcore_barrier()
    for t in range(RPS // B):
        row = sid * RPS + t * B
        pltpu.sync_copy(deg_sh.at[pl.ds(row, B)], zbuf_v)
        pltpu.sync_copy(zbuf_v, out_hbm.at[pl.ds(cid * NPAD + row, B)])


G = 8
CH0 = 160
CH1 = 2 * CH - CH0


@functools.partial(
    pl.kernel,
    out_type=jax.ShapeDtypeStruct((NC * NPAD, C), jnp.float32),
    mesh=_MESH,
    scratch_types=[
        pltpu.VMEM((G, B), jnp.int32),
        pltpu.VMEM((G, B), jnp.int32),
        pltpu.VMEM((B, C), jnp.float32),
        pltpu.VMEM((B, C), jnp.float32),
        pltpu.VMEM_SHARED((NPAD, C), jnp.float32),
        pltpu.SemaphoreType.DMA,
        pltpu.SemaphoreType.DMA,
    ],
)
def _agg_kernel(src_hbm, dst_hbm, z_hbm, out_hbm,
                src_v, dst_v, r0, r1, agg_sh, s0, s1):
    rows = (r0, r1)
    sems = (s0, s1)
    cid = lax.axis_index("c")
    sid = lax.axis_index("s")
    ngroups = jnp.where(cid == 0, CH0 // G, CH1 // G)
    chunk0 = jnp.where(cid == 0, sid * CH0, NS * CH0 + sid * CH1)
    pltpu.sync_copy(z_hbm.at[pl.ds(NPAD - B, B)], r0)
    for t in range(RPS // B):
        pltpu.sync_copy(r0, agg_sh.at[pl.ds(sid * RPS + t * B, B)])
    plsc.subcore_barrier()

    def group(gi, carry):
        base = chunk0 + gi * G
        pltpu.sync_copy(src_hbm.at[pl.ds(base, G)], src_v)
        pltpu.sync_copy(dst_hbm.at[pl.ds(base, G)], dst_v)
        h = [None] * G
        h[0] = pltpu.async_copy(z_hbm.at[src_v.at[0]], r0, s0)
        h[1] = pltpu.async_copy(z_hbm.at[src_v.at[1]], r1, s1)
        for t in range(G):
            h[t].wait()
            pltpu.sync_copy(rows[t % 2], agg_sh.at[dst_v.at[t]], add=True)
            if t + 2 < G:
                h[t + 2] = pltpu.async_copy(z_hbm.at[src_v.at[t + 2]],
                                            rows[t % 2], sems[t % 2])
        return carry

    lax.fori_loop(0, ngroups, group, 0)
    plsc.subcore_barrier()
    for t in range(RPS // B):
        row = sid * RPS + t * B
        pltpu.sync_copy(agg_sh.at[pl.ds(row, B)], r0)
        pltpu.sync_copy(r0, out_hbm.at[pl.ds(cid * NPAD + row, B)])


def _dinv_block(dp_ref):
    deg = (jnp.sum(dp_ref[0], axis=1, keepdims=True)
           + jnp.sum(dp_ref[1], axis=1, keepdims=True) + 1.0)
    rid = pl.program_id(0) * RB + lax.broadcasted_iota(jnp.int32, (RB, 1), 0)
    return jnp.where(rid < N, lax.rsqrt(deg), 0.0)


def _tc1(x_ref, dp_ref, g_ref, be_ref, mu_ref, va_ref, w_ref, z_ref):
    dinv = _dinv_block(dp_ref)
    s = g_ref[...] * lax.rsqrt(va_ref[...] + EPS)
    t = be_ref[...] - mu_ref[...] * s
    xb = x_ref[...] * s + t
    y = jnp.dot(xb, w_ref[...], preferred_element_type=jnp.float32)
    z_ref[...] = dinv * y


def _tc2(dp_ref, z1_ref, agg_ref, b1_ref, temb_ref, wt_ref, bt_ref,
         g_ref, be_ref, mu_ref, va_ref, w2_ref, z2_ref):
    dinv = _dinv_block(dp_ref)
    u = dinv * (agg_ref[0] + agg_ref[1] + z1_ref[...]) + b1_ref[...]
    h = u * jax.nn.sigmoid(u)
    tev = jnp.dot(temb_ref[...], wt_ref[...],
                  preferred_element_type=jnp.float32) + bt_ref[...]
    h = h + tev * jax.nn.sigmoid(tev)
    s = g_ref[...] * lax.rsqrt(va_ref[...] + EPS)
    t = be_ref[...] - mu_ref[...] * s
    hb = h * s + t
    z2_ref[...] = dinv * jnp.dot(hb, w2_ref[...],
                                 preferred_element_type=jnp.float32)


def _tc3(dp_ref, x_ref, z2_ref, agg_ref, b2_ref, o_ref):
    dinv = _dinv_block(dp_ref)
    u = dinv * (agg_ref[0] + agg_ref[1] + z2_ref[...]) + b2_ref[...]
    o_ref[...] = x_ref[...] + u * jax.nn.sigmoid(u)


_ROWS = pl.BlockSpec((RB, C), lambda i: (i, 0))
_DEGS = pl.BlockSpec((2, RB, DEGW), lambda i: (0, i, 0))
_AGGS = pl.BlockSpec((2, RB, C), lambda i: (0, i, 0))
_VEC = pl.BlockSpec((1, C), lambda i: (0, 0))
_MAT = pl.BlockSpec((C, C), lambda i: (0, 0))
_TE = pl.BlockSpec((1, DT), lambda i: (0, 0))
_WT = pl.BlockSpec((DT, C), lambda i: (0, 0))
_GRID = (NPAD // RB,)
_F32 = jnp.float32


def kernel(x, edge_index, t_emb, bn1_gamma, bn1_beta, bn1_mean, bn1_var,
           bn2_gamma, bn2_beta, bn2_mean, bn2_var, W1, b1, W2, b2, Wt, bt):
    pad = jnp.full((EPAD - E,), N, jnp.int32)
    src3 = jnp.concatenate([edge_index[0], pad]).reshape(NW * CH, B)
    dst3 = jnp.concatenate([edge_index[1], pad]).reshape(NW * CH, B)
    xp = jnp.pad(x, ((0, NPAD - N), (0, 0)))
    g1, be1, mu1, va1 = (a.reshape(1, C) for a in
                         (bn1_gamma, bn1_beta, bn1_mean, bn1_var))
    g2, be2, mu2, va2 = (a.reshape(1, C) for a in
                         (bn2_gamma, bn2_beta, bn2_mean, bn2_var))
    b1r, b2r, btr = b1.reshape(1, C), b2.reshape(1, C), bt.reshape(1, C)

    degp = _deg_kernel(dst3).reshape(2, NPAD, DEGW)

    z1 = pl.pallas_call(
        _tc1, grid=_GRID,
        in_specs=[_ROWS, _DEGS, _VEC, _VEC, _VEC, _VEC, _MAT],
        out_specs=_ROWS,
        out_shape=jax.ShapeDtypeStruct((NPAD, C), _F32),
    )(xp, degp, g1, be1, mu1, va1, W1)

    agg1 = _agg_kernel(src3, dst3, z1).reshape(2, NPAD, C)

    z2 = pl.pallas_call(
        _tc2, grid=_GRID,
        in_specs=[_DEGS, _ROWS, _AGGS, _VEC, _TE, _WT, _VEC,
                  _VEC, _VEC, _VEC, _VEC, _MAT],
        out_specs=_ROWS,
        out_shape=jax.ShapeDtypeStruct((NPAD, C), _F32),
    )(degp, z1, agg1, b1r, t_emb, Wt, btr, g2, be2, mu2, va2, W2)

    agg2 = _agg_kernel(src3, dst3, z2).reshape(2, NPAD, C)

    out = pl.pallas_call(
        _tc3, grid=_GRID,
        in_specs=[_DEGS, _ROWS, _ROWS, _AGGS, _VEC],
        out_specs=_ROWS,
        out_shape=jax.ShapeDtypeStruct((NPAD, C), _F32),
    )(degp, xp, z2, agg2, b2r)

    return out[:N]


# speedup vs baseline: 1.3596x; 1.3596x over previous
import functools

import jax
import jax.numpy as jnp
from jax import lax
from jax.experimental import pallas as pl
from jax.experimental.pallas import tpu as pltpu
from jax.experimental.pallas import tpu_sc as plsc

N = 10000
E = 320000
C = 128
DT = 256
EPS = 1e-5

NC = 2
NS = 16
NW = NC * NS
B = 128
CH = 80
TPW = CH * B
EPAD = NW * TPW
NPAD = 10240
RPS = NPAD // NS
DEGW = 16
RB = 1024

_MESH = plsc.VectorSubcoreMesh(
    core_axis_name="c", subcore_axis_name="s", num_cores=NC, num_subcores=NS)


@functools.partial(
    pl.kernel,
    out_type=jax.ShapeDtypeStruct((NC * NPAD, DEGW), jnp.float32),
    mesh=_MESH,
    scratch_types=[
        pltpu.VMEM((CH, B), jnp.int32),
        pltpu.VMEM((B, DEGW), jnp.float32),
        pltpu.VMEM((B, DEGW), jnp.float32),
        pltpu.VMEM_SHARED((NPAD, DEGW), jnp.float32),
    ],
)
def _deg_kernel(dst_hbm, out_hbm, dst_v, ones_v, zbuf_v, deg_sh):
    cid = lax.axis_index("c")
    sid = lax.axis_index("s")
    wid = cid * NS + sid
    pltpu.sync_copy(dst_hbm.at[pl.ds(wid * CH, CH)], dst_v)
    lane = lax.iota(jnp.int32, 16)
    one0 = jnp.where(lane == 0, 1.0, 0.0).astype(jnp.float32)
    zv = jnp.zeros((16,), jnp.float32)
    for r in range(B):
        ones_v[r, :] = one0
        zbuf_v[r, :] = zv
    for t in range(RPS // B):
        pltpu.sync_copy(zbuf_v, deg_sh.at[pl.ds(sid * RPS + t * B, B)])
    plsc.subcore_barrier()

    def body(j, carry):
        pltpu.sync_copy(ones_v, deg_sh.at[dst_v.at[j]], add=True)
        return carry

    lax.fori_loop(0, CH, body, 0)
    plsc.subcore_barrier()
    for t in range(RPS // B):
        row = sid * RPS + t * B
        pltpu.sync_copy(deg_sh.at[pl.ds(row, B)], zbuf_v)
        pltpu.sync_copy(zbuf_v, out_hbm.at[pl.ds(cid * NPAD + row, B)])


G = 8
CH0 = 136
CH1 = 2 * CH - CH0


@functools.partial(
    pl.kernel,
    out_type=jax.ShapeDtypeStruct((NC * NPAD, C), jnp.float32),
    mesh=_MESH,
    scratch_types=[
        pltpu.VMEM((G, B), jnp.int32),
        pltpu.VMEM((G, B), jnp.int32),
        pltpu.VMEM((B, C), jnp.float32),
        pltpu.VMEM((B, C), jnp.float32),
        pltpu.VMEM_SHARED((NPAD, C), jnp.float32),
        pltpu.SemaphoreType.DMA,
        pltpu.SemaphoreType.DMA,
    ],
)
def _agg_kernel(src_hbm, dst_hbm, z_hbm, out_hbm,
                src_v, dst_v, r0, r1, agg_sh, s0, s1):
    rows = (r0, r1)
    sems = (s0, s1)
    cid = lax.axis_index("c")
    sid = lax.axis_index("s")
    ngroups = jnp.where(cid == 0, CH0 // G, CH1 // G)
    chunk0 = jnp.where(cid == 0, sid * CH0, NS * CH0 + sid * CH1)
    pltpu.sync_copy(z_hbm.at[pl.ds(NPAD - B, B)], r0)
    for t in range(RPS // B):
        pltpu.sync_copy(r0, agg_sh.at[pl.ds(sid * RPS + t * B, B)])
    plsc.subcore_barrier()

    def group(gi, carry):
        base = chunk0 + gi * G
        pltpu.sync_copy(src_hbm.at[pl.ds(base, G)], src_v)
        pltpu.sync_copy(dst_hbm.at[pl.ds(base, G)], dst_v)
        h = [None] * G
        h[0] = pltpu.async_copy(z_hbm.at[src_v.at[0]], r0, s0)
        h[1] = pltpu.async_copy(z_hbm.at[src_v.at[1]], r1, s1)
        for t in range(G):
            h[t].wait()
            pltpu.sync_copy(rows[t % 2], agg_sh.at[dst_v.at[t]], add=True)
            if t + 2 < G:
                h[t + 2] = pltpu.async_copy(z_hbm.at[src_v.at[t + 2]],
                                            rows[t % 2], sems[t % 2])
        return carry

    lax.fori_loop(0, ngroups, group, 0)
    plsc.subcore_barrier()
    for t in range(RPS // B):
        row = sid * RPS + t * B
        pltpu.sync_copy(agg_sh.at[pl.ds(row, B)], r0)
        pltpu.sync_copy(r0, out_hbm.at[pl.ds(cid * NPAD + row, B)])


def _dinv_block(dp_ref):
    deg = (jnp.sum(dp_ref[0], axis=1, keepdims=True)
           + jnp.sum(dp_ref[1], axis=1, keepdims=True) + 1.0)
    rid = pl.program_id(0) * RB + lax.broadcasted_iota(jnp.int32, (RB, 1), 0)
    return jnp.where(rid < N, lax.rsqrt(deg), 0.0)


def _tc1(x_ref, dp_ref, g_ref, be_ref, mu_ref, va_ref, w_ref, z_ref):
    dinv = _dinv_block(dp_ref)
    s = g_ref[...] * lax.rsqrt(va_ref[...] + EPS)
    t = be_ref[...] - mu_ref[...] * s
    xb = x_ref[...] * s + t
    y = jnp.dot(xb, w_ref[...], preferred_element_type=jnp.float32)
    z_ref[...] = dinv * y


def _tc2(dp_ref, z1_ref, agg_ref, b1_ref, temb_ref, wt_ref, bt_ref,
         g_ref, be_ref, mu_ref, va_ref, w2_ref, z2_ref):
    dinv = _dinv_block(dp_ref)
    u = dinv * (agg_ref[0] + agg_ref[1] + z1_ref[...]) + b1_ref[...]
    h = u * jax.nn.sigmoid(u)
    tev = jnp.dot(temb_ref[...], wt_ref[...],
                  preferred_element_type=jnp.float32) + bt_ref[...]
    h = h + tev * jax.nn.sigmoid(tev)
    s = g_ref[...] * lax.rsqrt(va_ref[...] + EPS)
    t = be_ref[...] - mu_ref[...] * s
    hb = h * s + t
    z2_ref[...] = dinv * jnp.dot(hb, w2_ref[...],
                                 preferred_element_type=jnp.float32)


def _tc3(dp_ref, x_ref, z2_ref, agg_ref, b2_ref, o_ref):
    dinv = _dinv_block(dp_ref)
    u = dinv * (agg_ref[0] + agg_ref[1] + z2_ref[...]) + b2_ref[...]
    o_ref[...] = x_ref[...] + u * jax.nn.sigmoid(u)


_ROWS = pl.BlockSpec((RB, C), lambda i: (i, 0))
_DEGS = pl.BlockSpec((2, RB, DEGW), lambda i: (0, i, 0))
_AGGS = pl.BlockSpec((2, RB, C), lambda i: (0, i, 0))
_VEC = pl.BlockSpec((1, C), lambda i: (0, 0))
_MAT = pl.BlockSpec((C, C), lambda i: (0, 0))
_TE = pl.BlockSpec((1, DT), lambda i: (0, 0))
_WT = pl.BlockSpec((DT, C), lambda i: (0, 0))
_GRID = (NPAD // RB,)
_F32 = jnp.float32


def kernel(x, edge_index, t_emb, bn1_gamma, bn1_beta, bn1_mean, bn1_var,
           bn2_gamma, bn2_beta, bn2_mean, bn2_var, W1, b1, W2, b2, Wt, bt):
    pad = jnp.full((EPAD - E,), N, jnp.int32)
    src3 = jnp.concatenate([edge_index[0], pad]).reshape(NW * CH, B)
    dst3 = jnp.concatenate([edge_index[1], pad]).reshape(NW * CH, B)
    xp = jnp.pad(x, ((0, NPAD - N), (0, 0)))
    g1, be1, mu1, va1 = (a.reshape(1, C) for a in
                         (bn1_gamma, bn1_beta, bn1_mean, bn1_var))
    g2, be2, mu2, va2 = (a.reshape(1, C) for a in
                         (bn2_gamma, bn2_beta, bn2_mean, bn2_var))
    b1r, b2r, btr = b1.reshape(1, C), b2.reshape(1, C), bt.reshape(1, C)

    degp = _deg_kernel(dst3).reshape(2, NPAD, DEGW)

    z1 = pl.pallas_call(
        _tc1, grid=_GRID,
        in_specs=[_ROWS, _DEGS, _VEC, _VEC, _VEC, _VEC, _MAT],
        out_specs=_ROWS,
        out_shape=jax.ShapeDtypeStruct((NPAD, C), _F32),
    )(xp, degp, g1, be1, mu1, va1, W1)

    agg1 = _agg_kernel(src3, dst3, z1).reshape(2, NPAD, C)

    z2 = pl.pallas_call(
        _tc2, grid=_GRID,
        in_specs=[_DEGS, _ROWS, _AGGS, _VEC, _TE, _WT, _VEC,
                  _VEC, _VEC, _VEC, _VEC, _MAT],
        out_specs=_ROWS,
        out_shape=jax.ShapeDtypeStruct((NPAD, C), _F32),
    )(degp, z1, agg1, b1r, t_emb, Wt, btr, g2, be2, mu2, va2, W2)

    agg2 = _agg_kernel(src3, dst3, z2).reshape(2, NPAD, C)

    out = pl.pallas_call(
        _tc3, grid=_GRID,
        in_specs=[_DEGS, _ROWS, _ROWS, _AGGS, _VEC],
        out_specs=_ROWS,
        out_shape=jax.ShapeDtypeStruct((NPAD, C), _F32),
    )(degp, xp, z2, agg2, b2r)

    return out[:N]
